# baseline (speedup 1.0000x reference)
import functools

import jax
import jax.numpy as jnp
from jax import lax
from jax.experimental import pallas as pl
from jax.experimental.pallas import tpu as pltpu
from jax.experimental.pallas import tpu_sc as plsc

_NUM_CORES = 2
_NUM_SUBCORES = 16
_NW = _NUM_CORES * _NUM_SUBCORES

_SUB = 8
_CHUNK_R = 128
_NSUB = _CHUNK_R // _SUB
_CB = 512
_NSLICE = 4


def _sc_gather(w0_t, wf_t, b0, idx):
  B = idx.shape[0]
  bpw = B // _NW
  mesh = plsc.VectorSubcoreMesh(core_axis_name="c", subcore_axis_name="s")

  @functools.partial(
      pl.kernel,
      mesh=mesh,
      out_type=(
          jax.ShapeDtypeStruct((B, w0_t.shape[1]), jnp.float32),
          jax.ShapeDtypeStruct((B, wf_t.shape[1]), jnp.float32),
          jax.ShapeDtypeStruct((B, b0.shape[1]), jnp.float32),
      ),
      scratch_types=[
          pltpu.VMEM((bpw,), jnp.int32),
          pltpu.VMEM((bpw, w0_t.shape[1]), jnp.float32),
          pltpu.VMEM((bpw, wf_t.shape[1]), jnp.float32),
          pltpu.VMEM((bpw, b0.shape[1]), jnp.float32),
          pltpu.SemaphoreType.DMA,
          pltpu.SemaphoreType.DMA,
          pltpu.SemaphoreType.DMA,
      ],
      compiler_params=pltpu.CompilerParams(use_tc_tiling_on_sc=False),
  )
  def gather_kernel(w0_hbm, wf_hbm, b0_hbm, idx_hbm,
                    wg_hbm, wfg_hbm, bg_hbm,
                    idx_v, w_v, wf_v, b_v, sem0, sem1, sem2):
    wid = lax.axis_index("s") * _NUM_CORES + lax.axis_index("c")
    base = wid * bpw
    pltpu.sync_copy(idx_hbm.at[pl.ds(base, bpw)], idx_v)
    cp0 = pltpu.async_copy(w0_hbm.at[idx_v], w_v, sem0)
    cp1 = pltpu.async_copy(wf_hbm.at[idx_v], wf_v, sem1)
    cp2 = pltpu.async_copy(b0_hbm.at[idx_v], b_v, sem2)
    cp0.wait()
    cp1.wait()
    cp2.wait()
    pltpu.sync_copy(w_v, wg_hbm.at[pl.ds(base, bpw)])
    pltpu.sync_copy(wf_v, wfg_hbm.at[pl.ds(base, bpw)])
    pltpu.sync_copy(b_v, bg_hbm.at[pl.ds(base, bpw)])

  return gather_kernel(w0_t, wf_t, b0, idx)


def _dense_body(x_ref, wg_ref, wf_ref, b_ref, out_ref, wbd_ref, s_ref):
  k = pl.program_id(0)
  cb = pl.program_id(1)
  W = _SUB * 16

  @pl.when((k == 0) & (cb == 0))
  def _build_selectors():
    row = lax.broadcasted_iota(jnp.int32, (W, _CHUNK_R), 0)
    col = lax.broadcasted_iota(jnp.int32, (W, _CHUNK_R), 1)
    for j in range(_NSUB):
      s_ref[j] = jnp.where(col == j * _SUB + row // 16, 1.0, 0.0).astype(
          jnp.float32)

  @pl.when(cb == 0)
  def _build_block_diag():
    e_i = lax.broadcasted_iota(jnp.int32, (16, W), 0)
    c_i = lax.broadcasted_iota(jnp.int32, (16, W), 1)
    rep = jnp.where(c_i % 16 == e_i, 1.0, 0.0).astype(jnp.float32)
    rr = lax.broadcasted_iota(jnp.int32, (W, W), 0)
    cc = lax.broadcasted_iota(jnp.int32, (W, W), 1)
    msk = jnp.where(rr // 16 == cc // 16, 1.0, 0.0).astype(jnp.float32)
    for j in range(_NSUB):
      a = wg_ref[j * W:(j + 1) * W, :]
      wbd_ref[j] = lax.dot(a, rep, preferred_element_type=jnp.float32) * msk

  inv_sqrt2 = 0.7071067811865476
  acc = jnp.zeros((_CB, _CHUNK_R), jnp.float32)
  for j in range(_NSUB):
    xj = x_ref[:, j * W:(j + 1) * W]
    h = lax.dot(xj, wbd_ref[j], preferred_element_type=jnp.float32)
    h = h + b_ref[0, :, j * W:(j + 1) * W]
    h = 0.5 * h * (1.0 + lax.erf(h * inv_sqrt2))
    p = h * wf_ref[0, :, j * W:(j + 1) * W]
    acc = acc + lax.dot(p, s_ref[j], preferred_element_type=jnp.float32)
  out_ref[...] = acc


def _dense(x2, wg3, wff, bf, C, R):
  n_chunks = R // _CHUNK_R
  n_cb = C // _CB
  grid = (n_chunks, n_cb)
  return pl.pallas_call(
      _dense_body,
      grid=grid,
      in_specs=[
          pl.BlockSpec((_CB, _CHUNK_R * 16), lambda k, cb: (cb, k)),
          pl.BlockSpec((_CHUNK_R * 16, 16), lambda k, cb: (k, 0)),
          pl.BlockSpec((1, 1, _CHUNK_R * 16), lambda k, cb: (k, 0, 0)),
          pl.BlockSpec((1, 1, _CHUNK_R * 16), lambda k, cb: (k, 0, 0)),
      ],
      out_specs=pl.BlockSpec((_CB, _CHUNK_R), lambda k, cb: (cb, k)),
      out_shape=jax.ShapeDtypeStruct((C, R), jnp.float32),
      scratch_shapes=[
          pltpu.VMEM((_NSUB, _SUB * 16, _SUB * 16), jnp.float32),
          pltpu.VMEM((_NSUB, _SUB * 16, _CHUNK_R), jnp.float32),
      ],
      compiler_params=pltpu.CompilerParams(
          dimension_semantics=("arbitrary", "arbitrary"),
      ),
  )(x2, wg3, wff, bf)


def kernel(cell_region_embedding, regions_oi, W0, b0, Wf):
  C, R, D = cell_region_embedding.shape
  N = W0.shape[0]
  idx = regions_oi.astype(jnp.int32)

  w0_t = W0.reshape(N, D * D)
  wf_t = Wf[:, :, 0]
  wg, wfg, bg = _sc_gather(w0_t, wf_t, b0, idx)

  wg3 = wg.reshape(R * D, D)
  wff = wfg.reshape(R // _CHUNK_R, 1, _CHUNK_R * D)
  bf = bg.reshape(R // _CHUNK_R, 1, _CHUNK_R * D)

  cs = C // _NSLICE
  outs = []
  for i in range(_NSLICE):
    xs = lax.slice_in_dim(cell_region_embedding, i * cs, (i + 1) * cs, axis=0)
    x2s = xs.reshape(cs, R * D)
    outs.append(_dense(x2s, wg3, wff, bf, cs, R))
  return jnp.concatenate(outs, axis=0)


# speedup vs baseline: 1.0229x; 1.0229x over previous
import functools

import jax
import jax.numpy as jnp
from jax import lax
from jax.experimental import pallas as pl
from jax.experimental.pallas import tpu as pltpu
from jax.experimental.pallas import tpu_sc as plsc

_NUM_CORES = 2
_NUM_SUBCORES = 16
_NW = _NUM_CORES * _NUM_SUBCORES

_SUB = 8
_CHUNK_R = 128
_NSUB = _CHUNK_R // _SUB
_CB = 512
_NSLICE = 2


def _sc_gather(w0_t, wf_t, b0, idx):
  B = idx.shape[0]
  bpw = B // _NW
  mesh = plsc.VectorSubcoreMesh(core_axis_name="c", subcore_axis_name="s")

  @functools.partial(
      pl.kernel,
      mesh=mesh,
      out_type=(
          jax.ShapeDtypeStruct((B, w0_t.shape[1]), jnp.float32),
          jax.ShapeDtypeStruct((B, wf_t.shape[1]), jnp.float32),
          jax.ShapeDtypeStruct((B, b0.shape[1]), jnp.float32),
      ),
      scratch_types=[
          pltpu.VMEM((bpw,), jnp.int32),
          pltpu.VMEM((bpw, w0_t.shape[1]), jnp.float32),
          pltpu.VMEM((bpw, wf_t.shape[1]), jnp.float32),
          pltpu.VMEM((bpw, b0.shape[1]), jnp.float32),
          pltpu.SemaphoreType.DMA,
          pltpu.SemaphoreType.DMA,
          pltpu.SemaphoreType.DMA,
      ],
      compiler_params=pltpu.CompilerParams(use_tc_tiling_on_sc=False),
  )
  def gather_kernel(w0_hbm, wf_hbm, b0_hbm, idx_hbm,
                    wg_hbm, wfg_hbm, bg_hbm,
                    idx_v, w_v, wf_v, b_v, sem0, sem1, sem2):
    wid = lax.axis_index("s") * _NUM_CORES + lax.axis_index("c")
    base = wid * bpw
    pltpu.sync_copy(idx_hbm.at[pl.ds(base, bpw)], idx_v)
    cp0 = pltpu.async_copy(w0_hbm.at[idx_v], w_v, sem0)
    cp1 = pltpu.async_copy(wf_hbm.at[idx_v], wf_v, sem1)
    cp2 = pltpu.async_copy(b0_hbm.at[idx_v], b_v, sem2)
    cp0.wait()
    cp1.wait()
    cp2.wait()
    pltpu.sync_copy(w_v, wg_hbm.at[pl.ds(base, bpw)])
    pltpu.sync_copy(wf_v, wfg_hbm.at[pl.ds(base, bpw)])
    pltpu.sync_copy(b_v, bg_hbm.at[pl.ds(base, bpw)])

  return gather_kernel(w0_t, wf_t, b0, idx)


def _dense_body(x_ref, wg_ref, wf_ref, b_ref, out_ref, wbd_ref, s_ref):
  k = pl.program_id(0)
  cb = pl.program_id(1)
  W = _SUB * 16

  @pl.when((k == 0) & (cb == 0))
  def _build_selectors():
    row = lax.broadcasted_iota(jnp.int32, (W, _CHUNK_R), 0)
    col = lax.broadcasted_iota(jnp.int32, (W, _CHUNK_R), 1)
    for j in range(_NSUB):
      s_ref[j] = jnp.where(col == j * _SUB + row // 16, 1.0, 0.0).astype(
          jnp.float32)

  @pl.when(cb == 0)
  def _build_block_diag():
    e_i = lax.broadcasted_iota(jnp.int32, (16, W), 0)
    c_i = lax.broadcasted_iota(jnp.int32, (16, W), 1)
    rep = jnp.where(c_i % 16 == e_i, 1.0, 0.0).astype(jnp.float32)
    rr = lax.broadcasted_iota(jnp.int32, (W, W), 0)
    cc = lax.broadcasted_iota(jnp.int32, (W, W), 1)
    msk = jnp.where(rr // 16 == cc // 16, 1.0, 0.0).astype(jnp.float32)
    for j in range(_NSUB):
      a = wg_ref[j * W:(j + 1) * W, :]
      wbd_ref[j] = lax.dot(a, rep, preferred_element_type=jnp.float32) * msk

  inv_sqrt2 = 0.7071067811865476
  acc = jnp.zeros((_CB, _CHUNK_R), jnp.float32)
  for j in range(_NSUB):
    xj = x_ref[:, j * W:(j + 1) * W]
    h = lax.dot(xj, wbd_ref[j], preferred_element_type=jnp.float32)
    h = h + b_ref[0, :, j * W:(j + 1) * W]
    h = 0.5 * h * (1.0 + lax.erf(h * inv_sqrt2))
    p = h * wf_ref[0, :, j * W:(j + 1) * W]
    acc = acc + lax.dot(p, s_ref[j], preferred_element_type=jnp.float32)
  out_ref[...] = acc


def _dense(x2, wg3, wff, bf, C, R):
  n_chunks = R // _CHUNK_R
  n_cb = C // _CB
  grid = (n_chunks, n_cb)
  return pl.pallas_call(
      _dense_body,
      grid=grid,
      in_specs=[
          pl.BlockSpec((_CB, _CHUNK_R * 16), lambda k, cb: (cb, k)),
          pl.BlockSpec((_CHUNK_R * 16, 16), lambda k, cb: (k, 0)),
          pl.BlockSpec((1, 1, _CHUNK_R * 16), lambda k, cb: (k, 0, 0)),
          pl.BlockSpec((1, 1, _CHUNK_R * 16), lambda k, cb: (k, 0, 0)),
      ],
      out_specs=pl.BlockSpec((_CB, _CHUNK_R), lambda k, cb: (cb, k)),
      out_shape=jax.ShapeDtypeStruct((C, R), jnp.float32),
      scratch_shapes=[
          pltpu.VMEM((_NSUB, _SUB * 16, _SUB * 16), jnp.float32),
          pltpu.VMEM((_NSUB, _SUB * 16, _CHUNK_R), jnp.float32),
      ],
      compiler_params=pltpu.CompilerParams(
          dimension_semantics=("arbitrary", "arbitrary"),
      ),
  )(x2, wg3, wff, bf)


def kernel(cell_region_embedding, regions_oi, W0, b0, Wf):
  C, R, D = cell_region_embedding.shape
  N = W0.shape[0]
  idx = regions_oi.astype(jnp.int32)

  w0_t = W0.reshape(N, D * D)
  wf_t = Wf[:, :, 0]
  wg, wfg, bg = _sc_gather(w0_t, wf_t, b0, idx)

  wg3 = wg.reshape(R * D, D)
  wff = wfg.reshape(R // _CHUNK_R, 1, _CHUNK_R * D)
  bf = bg.reshape(R // _CHUNK_R, 1, _CHUNK_R * D)

  cs = C // _NSLICE
  outs = []
  for i in range(_NSLICE):
    xs = lax.slice_in_dim(cell_region_embedding, i * cs, (i + 1) * cs, axis=0)
    x2s = xs.reshape(cs, R * D)
    outs.append(_dense(x2s, wg3, wff, bf, cs, R))
  return jnp.concatenate(outs, axis=0)


# speedup vs baseline: 1.8871x; 1.8448x over previous
import functools

import jax
import jax.numpy as jnp
from jax import lax
from jax.experimental import pallas as pl
from jax.experimental.pallas import tpu as pltpu
from jax.experimental.pallas import tpu_sc as plsc

_NUM_CORES = 2
_NUM_SUBCORES = 16
_NW = _NUM_CORES * _NUM_SUBCORES

_SUB = 8
_CHUNK_R = 128
_NSUB = _CHUNK_R // _SUB
_CB = 1024


def _sc_gather(w0_t, wf_t, b0, idx):
  B = idx.shape[0]
  bpw = B // _NW
  mesh = plsc.VectorSubcoreMesh(core_axis_name="c", subcore_axis_name="s")

  @functools.partial(
      pl.kernel,
      mesh=mesh,
      out_type=(
          jax.ShapeDtypeStruct((B, w0_t.shape[1]), jnp.float32),
          jax.ShapeDtypeStruct((B, wf_t.shape[1]), jnp.float32),
          jax.ShapeDtypeStruct((B, b0.shape[1]), jnp.float32),
      ),
      scratch_types=[
          pltpu.VMEM((bpw,), jnp.int32),
          pltpu.VMEM((bpw, w0_t.shape[1]), jnp.float32),
          pltpu.VMEM((bpw, wf_t.shape[1]), jnp.float32),
          pltpu.VMEM((bpw, b0.shape[1]), jnp.float32),
          pltpu.SemaphoreType.DMA,
          pltpu.SemaphoreType.DMA,
          pltpu.SemaphoreType.DMA,
      ],
      compiler_params=pltpu.CompilerParams(use_tc_tiling_on_sc=False),
  )
  def gather_kernel(w0_hbm, wf_hbm, b0_hbm, idx_hbm,
                    wg_hbm, wfg_hbm, bg_hbm,
                    idx_v, w_v, wf_v, b_v, sem0, sem1, sem2):
    wid = lax.axis_index("s") * _NUM_CORES + lax.axis_index("c")
    base = wid * bpw
    pltpu.sync_copy(idx_hbm.at[pl.ds(base, bpw)], idx_v)
    cp0 = pltpu.async_copy(w0_hbm.at[idx_v], w_v, sem0)
    cp1 = pltpu.async_copy(wf_hbm.at[idx_v], wf_v, sem1)
    cp2 = pltpu.async_copy(b0_hbm.at[idx_v], b_v, sem2)
    cp0.wait()
    cp1.wait()
    cp2.wait()
    pltpu.sync_copy(w_v, wg_hbm.at[pl.ds(base, bpw)])
    pltpu.sync_copy(wf_v, wfg_hbm.at[pl.ds(base, bpw)])
    pltpu.sync_copy(b_v, bg_hbm.at[pl.ds(base, bpw)])

  return gather_kernel(w0_t, wf_t, b0, idx)


def _dense_body(x_ref, wg_ref, wf_ref, b_ref, out_ref, wbd_ref, s_ref, bc_ref):
  cb = pl.program_id(1)
  W = _SUB * 16

  @pl.when(cb == 0)
  def _build():
    e_i = lax.broadcasted_iota(jnp.int32, (16, W), 0)
    c_i = lax.broadcasted_iota(jnp.int32, (16, W), 1)
    rep = jnp.where(c_i % 16 == e_i, 1.0, 0.0).astype(jnp.float32)
    rr = lax.broadcasted_iota(jnp.int32, (W, W), 0)
    cc = lax.broadcasted_iota(jnp.int32, (W, W), 1)
    msk = jnp.where(rr // 16 == cc // 16, 1.0, 0.0).astype(jnp.float32)
    for j in range(_NSUB):
      a = wg_ref[j * W:(j + 1) * W, :]
      wbd_ref[j] = lax.dot(a, rep, preferred_element_type=jnp.float32) * msk
      wfrow = wf_ref[0, :, j * W:(j + 1) * W]
      wfb = jnp.broadcast_to(wfrow, (W, W))
      qsel = cc // 16 == rr - j * _SUB
      s_ref[j] = jnp.where(qsel, wfb, 0.0).astype(jnp.float32)
      brow = b_ref[0, :, j * W:(j + 1) * W]
      bb = jnp.broadcast_to(brow, (W, W))
      dmat = jnp.where(rr == cc, bb, 0.0).astype(jnp.float32)
      bc_ref[j * W:(j + 1) * W, :] = jnp.sum(dmat, axis=1, keepdims=True)

  inv_sqrt2 = 0.7071067811865476
  x2 = x_ref[...].reshape(_CHUNK_R * 16, _CB)
  acc = jnp.zeros((_CHUNK_R, _CB), jnp.float32)
  for j in range(_NSUB):
    xg = x2[j * W:(j + 1) * W, :]
    h = lax.dot(wbd_ref[j], xg, preferred_element_type=jnp.float32)
    h = h + bc_ref[j * W:(j + 1) * W, :]
    h = 0.5 * h * (1.0 + lax.erf(h * inv_sqrt2))
    acc = acc + lax.dot(s_ref[j], h, preferred_element_type=jnp.float32)
  out_ref[...] = acc


def _dense(xq, wgt, wff, bf, C, R):
  n_chunks = R // _CHUNK_R
  n_cb = C // _CB
  grid = (n_chunks, n_cb)
  return pl.pallas_call(
      _dense_body,
      grid=grid,
      in_specs=[
          pl.BlockSpec((_CHUNK_R, 16, _CB), lambda k, cb: (k, 0, cb)),
          pl.BlockSpec((_CHUNK_R * 16, 16), lambda k, cb: (k, 0)),
          pl.BlockSpec((1, 1, _CHUNK_R * 16), lambda k, cb: (k, 0, 0)),
          pl.BlockSpec((1, 1, _CHUNK_R * 16), lambda k, cb: (k, 0, 0)),
      ],
      out_specs=pl.BlockSpec((_CHUNK_R, _CB), lambda k, cb: (k, cb)),
      out_shape=jax.ShapeDtypeStruct((R, C), jnp.float32),
      scratch_shapes=[
          pltpu.VMEM((_NSUB, _SUB * 16, _SUB * 16), jnp.float32),
          pltpu.VMEM((_NSUB, _SUB * 16, _CHUNK_R), jnp.float32),
          pltpu.VMEM((_CHUNK_R * 16, 1), jnp.float32),
      ],
      compiler_params=pltpu.CompilerParams(
          dimension_semantics=("arbitrary", "arbitrary"),
      ),
  )(xq, wgt, wff, bf)


def kernel(cell_region_embedding, regions_oi, W0, b0, Wf):
  C, R, D = cell_region_embedding.shape
  N = W0.shape[0]
  idx = regions_oi.astype(jnp.int32)

  w0_t = W0.reshape(N, D * D)
  wf_t = Wf[:, :, 0]
  wg, wfg, bg = _sc_gather(w0_t, wf_t, b0, idx)

  xq = jnp.transpose(cell_region_embedding, (1, 2, 0))
  wgt = jnp.transpose(wg.reshape(R, D, D), (0, 2, 1)).reshape(R * D, D)
  wff = wfg.reshape(R // _CHUNK_R, 1, _CHUNK_R * D)
  bf = bg.reshape(R // _CHUNK_R, 1, _CHUNK_R * D)

  out_t = _dense(xq, wgt, wff, bf, C, R)
  return jnp.transpose(out_t, (1, 0))


# speedup vs baseline: 1.9319x; 1.0237x over previous
import functools

import jax
import jax.numpy as jnp
from jax import lax
from jax.experimental import pallas as pl
from jax.experimental.pallas import tpu as pltpu
from jax.experimental.pallas import tpu_sc as plsc

_NUM_CORES = 2
_NUM_SUBCORES = 16
_NW = _NUM_CORES * _NUM_SUBCORES

_SUB = 8
_CHUNK_R = 128
_NSUB = _CHUNK_R // _SUB
_CB = 1024


def _sc_gather(w0_t, wf_t, b0, idx):
  B = idx.shape[0]
  bpw = B // _NW
  mesh = plsc.VectorSubcoreMesh(core_axis_name="c", subcore_axis_name="s")

  @functools.partial(
      pl.kernel,
      mesh=mesh,
      out_type=(
          jax.ShapeDtypeStruct((B, w0_t.shape[1]), jnp.float32),
          jax.ShapeDtypeStruct((B, wf_t.shape[1]), jnp.float32),
          jax.ShapeDtypeStruct((B, b0.shape[1]), jnp.float32),
      ),
      scratch_types=[
          pltpu.VMEM((bpw,), jnp.int32),
          pltpu.VMEM((bpw, w0_t.shape[1]), jnp.float32),
          pltpu.VMEM((bpw, wf_t.shape[1]), jnp.float32),
          pltpu.VMEM((bpw, b0.shape[1]), jnp.float32),
          pltpu.SemaphoreType.DMA,
          pltpu.SemaphoreType.DMA,
          pltpu.SemaphoreType.DMA,
      ],
      compiler_params=pltpu.CompilerParams(use_tc_tiling_on_sc=False),
  )
  def gather_kernel(w0_hbm, wf_hbm, b0_hbm, idx_hbm,
                    wg_hbm, wfg_hbm, bg_hbm,
                    idx_v, w_v, wf_v, b_v, sem0, sem1, sem2):
    wid = lax.axis_index("s") * _NUM_CORES + lax.axis_index("c")
    base = wid * bpw
    pltpu.sync_copy(idx_hbm.at[pl.ds(base, bpw)], idx_v)
    cp0 = pltpu.async_copy(w0_hbm.at[idx_v], w_v, sem0)
    cp1 = pltpu.async_copy(wf_hbm.at[idx_v], wf_v, sem1)
    cp2 = pltpu.async_copy(b0_hbm.at[idx_v], b_v, sem2)
    cp0.wait()
    cp1.wait()
    cp2.wait()
    pltpu.sync_copy(w_v, wg_hbm.at[pl.ds(base, bpw)])
    pltpu.sync_copy(wf_v, wfg_hbm.at[pl.ds(base, bpw)])
    pltpu.sync_copy(b_v, bg_hbm.at[pl.ds(base, bpw)])

  return gather_kernel(w0_t, wf_t, b0, idx)


def _dense_body(x_ref, wg_ref, wf_ref, b_ref, out_ref, wbd_ref, s_ref, bc_ref):
  cb = pl.program_id(1)
  W = _SUB * 16

  @pl.when(cb == 0)
  def _build():
    e_i = lax.broadcasted_iota(jnp.int32, (16, W), 0)
    c_i = lax.broadcasted_iota(jnp.int32, (16, W), 1)
    rep = jnp.where(c_i % 16 == e_i, 1.0, 0.0).astype(jnp.float32)
    rr = lax.broadcasted_iota(jnp.int32, (W, W), 0)
    cc = lax.broadcasted_iota(jnp.int32, (W, W), 1)
    msk = jnp.where(rr // 16 == cc // 16, 1.0, 0.0).astype(jnp.float32)
    for j in range(_NSUB):
      a = wg_ref[j * W:(j + 1) * W, :]
      wbd_ref[j] = lax.dot(a, rep, preferred_element_type=jnp.float32) * msk
      wfrow = wf_ref[0, :, j * W:(j + 1) * W]
      wfb = jnp.broadcast_to(wfrow, (W, W))
      qsel = cc // 16 == rr - j * _SUB
      s_ref[j] = jnp.where(qsel, wfb, 0.0).astype(jnp.float32)
      brow = b_ref[0, :, j * W:(j + 1) * W]
      bb = jnp.broadcast_to(brow, (W, W))
      dmat = jnp.where(rr == cc, bb, 0.0).astype(jnp.float32)
      bc_ref[j * W:(j + 1) * W, :] = jnp.sum(dmat, axis=1, keepdims=True)

  inv_sqrt2 = 0.7071067811865476
  x2 = x_ref[...].reshape(_CHUNK_R * 16, _CB)
  acc = jnp.zeros((_CHUNK_R, _CB), jnp.float32)
  for j in range(_NSUB):
    xg = x2[j * W:(j + 1) * W, :]
    h = lax.dot(wbd_ref[j], xg, preferred_element_type=jnp.float32)
    h = h + bc_ref[j * W:(j + 1) * W, :]
    h = 0.5 * h * (1.0 + lax.erf(h * inv_sqrt2))
    acc = acc + lax.dot(s_ref[j], h, preferred_element_type=jnp.float32)
  out_ref[...] = lax.transpose(acc, (1, 0))


def _dense(xq, wgt, wff, bf, C, R):
  n_chunks = R // _CHUNK_R
  n_cb = C // _CB
  grid = (n_chunks, n_cb)
  return pl.pallas_call(
      _dense_body,
      grid=grid,
      in_specs=[
          pl.BlockSpec((_CHUNK_R, 16, _CB), lambda k, cb: (k, 0, cb)),
          pl.BlockSpec((_CHUNK_R * 16, 16), lambda k, cb: (k, 0)),
          pl.BlockSpec((1, 1, _CHUNK_R * 16), lambda k, cb: (k, 0, 0)),
          pl.BlockSpec((1, 1, _CHUNK_R * 16), lambda k, cb: (k, 0, 0)),
      ],
      out_specs=pl.BlockSpec((_CB, _CHUNK_R), lambda k, cb: (cb, k)),
      out_shape=jax.ShapeDtypeStruct((C, R), jnp.float32),
      scratch_shapes=[
          pltpu.VMEM((_NSUB, _SUB * 16, _SUB * 16), jnp.float32),
          pltpu.VMEM((_NSUB, _SUB * 16, _CHUNK_R), jnp.float32),
          pltpu.VMEM((_CHUNK_R * 16, 1), jnp.float32),
      ],
      compiler_params=pltpu.CompilerParams(
          dimension_semantics=("arbitrary", "arbitrary"),
      ),
  )(xq, wgt, wff, bf)


def kernel(cell_region_embedding, regions_oi, W0, b0, Wf):
  C, R, D = cell_region_embedding.shape
  N = W0.shape[0]
  idx = regions_oi.astype(jnp.int32)

  w0_t = W0.reshape(N, D * D)
  wf_t = Wf[:, :, 0]
  wg, wfg, bg = _sc_gather(w0_t, wf_t, b0, idx)

  xq = jnp.transpose(cell_region_embedding, (1, 2, 0))
  wgt = jnp.transpose(wg.reshape(R, D, D), (0, 2, 1)).reshape(R * D, D)
  wff = wfg.reshape(R // _CHUNK_R, 1, _CHUNK_R * D)
  bf = bg.reshape(R // _CHUNK_R, 1, _CHUNK_R * D)

  return _dense(xq, wgt, wff, bf, C, R)


# speedup vs baseline: 2.1279x; 1.1015x over previous
import functools

import jax
import jax.numpy as jnp
from jax import lax
from jax.experimental import pallas as pl
from jax.experimental.pallas import tpu as pltpu
from jax.experimental.pallas import tpu_sc as plsc

_NUM_CORES = 2
_NUM_SUBCORES = 16
_NW = _NUM_CORES * _NUM_SUBCORES

_SUB = 8
_CHUNK_R = 128
_NSUB = _CHUNK_R // _SUB
_CB = 2048


def _sc_gather(w0_t, wf_t, b0, idx):
  B = idx.shape[0]
  bpw = B // _NW
  mesh = plsc.VectorSubcoreMesh(core_axis_name="c", subcore_axis_name="s")

  @functools.partial(
      pl.kernel,
      mesh=mesh,
      out_type=(
          jax.ShapeDtypeStruct((B, w0_t.shape[1]), jnp.float32),
          jax.ShapeDtypeStruct((B, wf_t.shape[1]), jnp.float32),
          jax.ShapeDtypeStruct((B, b0.shape[1]), jnp.float32),
      ),
      scratch_types=[
          pltpu.VMEM((bpw,), jnp.int32),
          pltpu.VMEM((bpw, w0_t.shape[1]), jnp.float32),
          pltpu.VMEM((bpw, wf_t.shape[1]), jnp.float32),
          pltpu.VMEM((bpw, b0.shape[1]), jnp.float32),
          pltpu.SemaphoreType.DMA,
          pltpu.SemaphoreType.DMA,
          pltpu.SemaphoreType.DMA,
      ],
      compiler_params=pltpu.CompilerParams(use_tc_tiling_on_sc=False),
  )
  def gather_kernel(w0_hbm, wf_hbm, b0_hbm, idx_hbm,
                    wg_hbm, wfg_hbm, bg_hbm,
                    idx_v, w_v, wf_v, b_v, sem0, sem1, sem2):
    wid = lax.axis_index("s") * _NUM_CORES + lax.axis_index("c")
    base = wid * bpw
    pltpu.sync_copy(idx_hbm.at[pl.ds(base, bpw)], idx_v)
    cp0 = pltpu.async_copy(w0_hbm.at[idx_v], w_v, sem0)
    cp1 = pltpu.async_copy(wf_hbm.at[idx_v], wf_v, sem1)
    cp2 = pltpu.async_copy(b0_hbm.at[idx_v], b_v, sem2)
    cp0.wait()
    cp1.wait()
    cp2.wait()
    pltpu.sync_copy(w_v, wg_hbm.at[pl.ds(base, bpw)])
    pltpu.sync_copy(wf_v, wfg_hbm.at[pl.ds(base, bpw)])
    pltpu.sync_copy(b_v, bg_hbm.at[pl.ds(base, bpw)])

  return gather_kernel(w0_t, wf_t, b0, idx)


def _dense_body(x_ref, wg_ref, wf_ref, b_ref, out_ref, wbd_ref, s_ref, bc_ref):
  cb = pl.program_id(1)
  W = _SUB * 16

  @pl.when(cb == 0)
  def _build():
    e_i = lax.broadcasted_iota(jnp.int32, (16, W), 0)
    c_i = lax.broadcasted_iota(jnp.int32, (16, W), 1)
    rep = jnp.where(c_i % 16 == e_i, 1.0, 0.0).astype(jnp.float32)
    rr = lax.broadcasted_iota(jnp.int32, (W, W), 0)
    cc = lax.broadcasted_iota(jnp.int32, (W, W), 1)
    msk = jnp.where(rr // 16 == cc // 16, 1.0, 0.0).astype(jnp.float32)
    for j in range(_NSUB):
      a = wg_ref[j * W:(j + 1) * W, :]
      wbd_ref[j] = lax.dot(a, rep, preferred_element_type=jnp.float32) * msk
      wfrow = wf_ref[0, :, j * W:(j + 1) * W]
      wfb = jnp.broadcast_to(wfrow, (W, W))
      qsel = cc // 16 == rr - j * _SUB
      s_ref[j] = jnp.where(qsel, wfb, 0.0).astype(jnp.float32)
      brow = b_ref[0, :, j * W:(j + 1) * W]
      bb = jnp.broadcast_to(brow, (W, W))
      dmat = jnp.where(rr == cc, bb, 0.0).astype(jnp.float32)
      bc_ref[j * W:(j + 1) * W, :] = jnp.sum(dmat, axis=1, keepdims=True)

  inv_sqrt2 = 0.7071067811865476
  x2 = x_ref[...].reshape(_CHUNK_R * 16, _CB)
  acc = jnp.zeros((_CHUNK_R, _CB), jnp.float32)
  for j in range(_NSUB):
    xg = x2[j * W:(j + 1) * W, :]
    h = lax.dot(wbd_ref[j], xg, preferred_element_type=jnp.float32)
    h = h + bc_ref[j * W:(j + 1) * W, :]
    h = 0.5 * h * (1.0 + lax.erf(h * inv_sqrt2))
    acc = acc + lax.dot(s_ref[j], h, preferred_element_type=jnp.float32)
  out_ref[...] = lax.transpose(acc, (1, 0))


def _dense(xq, wgt, wff, bf, C, R):
  n_chunks = R // _CHUNK_R
  n_cb = C // _CB
  grid = (n_chunks, n_cb)
  return pl.pallas_call(
      _dense_body,
      grid=grid,
      in_specs=[
          pl.BlockSpec((_CHUNK_R, 16, _CB), lambda k, cb: (k, 0, cb)),
          pl.BlockSpec((_CHUNK_R * 16, 16), lambda k, cb: (k, 0)),
          pl.BlockSpec((1, 1, _CHUNK_R * 16), lambda k, cb: (k, 0, 0)),
          pl.BlockSpec((1, 1, _CHUNK_R * 16), lambda k, cb: (k, 0, 0)),
      ],
      out_specs=pl.BlockSpec((_CB, _CHUNK_R), lambda k, cb: (cb, k)),
      out_shape=jax.ShapeDtypeStruct((C, R), jnp.float32),
      scratch_shapes=[
          pltpu.VMEM((_NSUB, _SUB * 16, _SUB * 16), jnp.float32),
          pltpu.VMEM((_NSUB, _SUB * 16, _CHUNK_R), jnp.float32),
          pltpu.VMEM((_CHUNK_R * 16, 1), jnp.float32),
      ],
      compiler_params=pltpu.CompilerParams(
          dimension_semantics=("arbitrary", "arbitrary"),
      ),
  )(xq, wgt, wff, bf)


def kernel(cell_region_embedding, regions_oi, W0, b0, Wf):
  C, R, D = cell_region_embedding.shape
  N = W0.shape[0]
  idx = regions_oi.astype(jnp.int32)

  w0_t = W0.reshape(N, D * D)
  wf_t = Wf[:, :, 0]
  wg, wfg, bg = _sc_gather(w0_t, wf_t, b0, idx)

  xq = jnp.transpose(cell_region_embedding, (1, 2, 0))
  wgt = jnp.transpose(wg.reshape(R, D, D), (0, 2, 1)).reshape(R * D, D)
  wff = wfg.reshape(R // _CHUNK_R, 1, _CHUNK_R * D)
  bf = bg.reshape(R // _CHUNK_R, 1, _CHUNK_R * D)

  return _dense(xq, wgt, wff, bf, C, R)
